# baseline (speedup 1.0000x reference)
import functools

import jax
import jax.numpy as jnp
from jax import lax
from jax.experimental import pallas as pl
from jax.experimental.pallas import tpu as pltpu
from jax.experimental.pallas import tpu_sc as plsc


def _gather_sum(x3, table, B, CH, CL, D):
  info = plsc.get_sparse_core_info()
  NC, NS = info.num_cores, info.num_subcores
  NW = NC * NS
  assert B % NW == 0
  SPW = B // NW
  assert SPW % 2 == 0
  NV = D // 16

  mesh = plsc.VectorSubcoreMesh(core_axis_name="c", subcore_axis_name="s")

  @functools.partial(
      pl.kernel,
      mesh=mesh,
      out_type=jax.ShapeDtypeStruct((B, D), jnp.float32),
      scratch_types=[
          pltpu.VMEM((SPW * CH * CL,), jnp.int32),
          pltpu.VMEM((2, CL, D), jnp.float32),
          pltpu.VMEM((SPW, D), jnp.float32),
          pltpu.SemaphoreType.DMA((2,)),
      ],
  )
  def k(x_hbm, table_hbm, out_hbm, idx_v, rows_v, acc_v, sems):
    cid = lax.axis_index("c")
    sid = lax.axis_index("s")
    wid = sid * NC + cid
    base = wid * SPW

    pltpu.sync_copy(x_hbm.at[pl.ds(base * CH * CL, SPW * CH * CL)], idx_v)
    zero = jnp.zeros((16,), jnp.float32)

    def zero_buf(slot):
      def zb(r, c):
        for j in range(NV):
          rows_v[slot, r, pl.ds(j * 16, 16)] = zero
        return c

      lax.fori_loop(0, CL, zb, 0)

    zero_buf(0)
    zero_buf(1)

    def step(i, carry):
      slot = i % 2

      @pl.when(i >= 2)
      def _():
        si = i - 2

        def db(ch, c):
          pltpu.make_async_copy(
              table_hbm.at[idx_v.at[pl.ds(0, CL)]],
              rows_v.at[slot],
              sems.at[slot],
          ).wait()
          return c

        lax.fori_loop(0, CH, db, 0)

        def body(r, acc):
          out = []
          for j in range(NV):
            out.append(acc[j] + rows_v[slot, r, pl.ds(j * 16, 16)])
          for j in range(NV):
            rows_v[slot, r, pl.ds(j * 16, 16)] = zero
          return tuple(out)

        acc = lax.fori_loop(0, CL, body, (zero,) * NV)
        for j in range(NV):
          acc_v[si, pl.ds(j * 16, 16)] = acc[j]

      @pl.when(i < SPW)
      def _():
        def fb(ch, c):
          pltpu.async_copy(
              table_hbm.at[idx_v.at[pl.ds((i * CH + ch) * CL, CL)]],
              rows_v.at[slot],
              sems.at[slot],
              add=True,
          )
          return c

        lax.fori_loop(0, CH, fb, 0)

      return carry

    lax.fori_loop(0, SPW + 2, step, 0)
    pltpu.sync_copy(acc_v, out_hbm.at[pl.ds(base, SPW)])

  return k(x3, table)


def _mlp_body(rep_ref, len_ref, w1t_ref, b1_ref, w2t_ref, b2_ref, out_ref):
  rep = rep_ref[...] / len_ref[...]
  h = jnp.dot(rep, w1t_ref[...], preferred_element_type=jnp.float32)
  h = jnp.maximum(h + b1_ref[...], 0.0)
  out = jnp.dot(h, w2t_ref[...], preferred_element_type=jnp.float32)
  out_ref[...] = out + b2_ref[...]


def kernel(x, lengths, table, W1, b1, W2, b2):
  B, L = x.shape
  D = table.shape[1]
  H = W1.shape[0]
  O = W2.shape[0]

  CL = max((r for r in range(8, 65, 8) if L % r == 0), default=L)
  CH = L // CL
  x3 = x.reshape(B * L)

  rep_sum = _gather_sum(x3, table, B, CH, CL, D)

  lens = lengths.astype(jnp.float32).reshape(B, 1)
  logits = pl.pallas_call(
      _mlp_body,
      out_shape=jax.ShapeDtypeStruct((B, O), jnp.float32),
  )(rep_sum, lens, W1.T, b1.reshape(1, H), W2.T, b2.reshape(1, O))
  return logits


# speedup vs baseline: 1.1748x; 1.1748x over previous
import functools

import jax
import jax.numpy as jnp
from jax import lax
from jax.experimental import pallas as pl
from jax.experimental.pallas import tpu as pltpu
from jax.experimental.pallas import tpu_sc as plsc


def _gather_sum(x3, table, B, CH, CL, D):
  info = plsc.get_sparse_core_info()
  NC, NS = info.num_cores, info.num_subcores
  NW = NC * NS
  assert B % NW == 0
  SPW = B // NW
  assert SPW % 2 == 0
  NV = D // 16
  NSLOT = 4

  mesh = plsc.VectorSubcoreMesh(core_axis_name="c", subcore_axis_name="s")

  @functools.partial(
      pl.kernel,
      mesh=mesh,
      out_type=jax.ShapeDtypeStruct((B, D), jnp.float32),
      scratch_types=[
          pltpu.VMEM((SPW * CH * CL,), jnp.int32),
          pltpu.VMEM((NSLOT, CL, D), jnp.float32),
          pltpu.VMEM((SPW, D), jnp.float32),
          pltpu.SemaphoreType.DMA((NSLOT,)),
      ],
  )
  def k(x_hbm, table_hbm, out_hbm, idx_v, rows_v, acc_v, sems):
    cid = lax.axis_index("c")
    sid = lax.axis_index("s")
    wid = sid * NC + cid
    base = wid * SPW

    pltpu.sync_copy(x_hbm.at[pl.ds(base * CH * CL, SPW * CH * CL)], idx_v)
    zero = jnp.zeros((16,), jnp.float32)

    def zero_buf(slot):
      def zb(r, c):
        for j in range(NV):
          rows_v[slot, r, pl.ds(j * 16, 16)] = zero
        return c

      lax.fori_loop(0, CL, zb, 0)

    for sl in range(NSLOT):
      zero_buf(sl)

    def step(i, carry):
      slot = i % NSLOT

      @pl.when(i >= NSLOT)
      def _():
        si = i - NSLOT

        def db(ch, c):
          pltpu.make_async_copy(
              table_hbm.at[idx_v.at[pl.ds(0, CL)]],
              rows_v.at[slot],
              sems.at[slot],
          ).wait()
          return c

        lax.fori_loop(0, CH, db, 0)

        def body(r, acc):
          out = []
          for j in range(NV):
            out.append(acc[j] + rows_v[slot, r, pl.ds(j * 16, 16)])
          for j in range(NV):
            rows_v[slot, r, pl.ds(j * 16, 16)] = zero
          return tuple(out)

        acc = lax.fori_loop(0, CL, body, (zero,) * NV)
        for j in range(NV):
          acc_v[si, pl.ds(j * 16, 16)] = acc[j]

      @pl.when(i < SPW)
      def _():
        def fb(ch, c):
          pltpu.async_copy(
              table_hbm.at[idx_v.at[pl.ds((i * CH + ch) * CL, CL)]],
              rows_v.at[slot],
              sems.at[slot],
              add=True,
          )
          return c

        lax.fori_loop(0, CH, fb, 0)

      return carry

    lax.fori_loop(0, SPW + NSLOT, step, 0)
    pltpu.sync_copy(acc_v, out_hbm.at[pl.ds(base, SPW)])

  return k(x3, table)


def _mlp_body(rep_ref, len_ref, w1t_ref, b1_ref, w2t_ref, b2_ref, out_ref):
  rep = rep_ref[...] / len_ref[...]
  h = jnp.dot(rep, w1t_ref[...], preferred_element_type=jnp.float32)
  h = jnp.maximum(h + b1_ref[...], 0.0)
  out = jnp.dot(h, w2t_ref[...], preferred_element_type=jnp.float32)
  out_ref[...] = out + b2_ref[...]


def kernel(x, lengths, table, W1, b1, W2, b2):
  B, L = x.shape
  D = table.shape[1]
  H = W1.shape[0]
  O = W2.shape[0]

  CL = max((r for r in range(8, 65, 8) if L % r == 0), default=L)
  CH = L // CL
  x3 = x.reshape(B * L)

  rep_sum = _gather_sum(x3, table, B, CH, CL, D)

  lens = lengths.astype(jnp.float32).reshape(B, 1)
  logits = pl.pallas_call(
      _mlp_body,
      out_shape=jax.ShapeDtypeStruct((B, O), jnp.float32),
  )(rep_sum, lens, W1.T, b1.reshape(1, H), W2.T, b2.reshape(1, O))
  return logits


# speedup vs baseline: 1.2254x; 1.0430x over previous
import functools

import jax
import jax.numpy as jnp
from jax import lax
from jax.experimental import pallas as pl
from jax.experimental.pallas import tpu as pltpu
from jax.experimental.pallas import tpu_sc as plsc


def _gather_sum(x3, table, B, CH, CL, D):
  info = plsc.get_sparse_core_info()
  NC, NS = info.num_cores, info.num_subcores
  NW = NC * NS
  assert B % NW == 0
  SPW = B // NW
  assert SPW % 2 == 0
  NV = D // 16
  NSLOT = 8

  mesh = plsc.VectorSubcoreMesh(core_axis_name="c", subcore_axis_name="s")

  @functools.partial(
      pl.kernel,
      mesh=mesh,
      out_type=jax.ShapeDtypeStruct((B, D), jnp.float32),
      scratch_types=[
          pltpu.VMEM((SPW * CH * CL,), jnp.int32),
          pltpu.VMEM((NSLOT, CL, D), jnp.float32),
          pltpu.VMEM((SPW, D), jnp.float32),
          pltpu.SemaphoreType.DMA((NSLOT,)),
      ],
  )
  def k(x_hbm, table_hbm, out_hbm, idx_v, rows_v, acc_v, sems):
    cid = lax.axis_index("c")
    sid = lax.axis_index("s")
    wid = sid * NC + cid
    base = wid * SPW

    pltpu.sync_copy(x_hbm.at[pl.ds(base * CH * CL, SPW * CH * CL)], idx_v)
    zero = jnp.zeros((16,), jnp.float32)

    def zero_buf(slot):
      def zb(r, c):
        for j in range(NV):
          rows_v[slot, r, pl.ds(j * 16, 16)] = zero
        return c

      lax.fori_loop(0, CL, zb, 0)

    for sl in range(NSLOT):
      zero_buf(sl)

    def step(i, carry):
      slot = i % NSLOT

      @pl.when(i >= NSLOT)
      def _():
        si = i - NSLOT

        def db(ch, c):
          pltpu.make_async_copy(
              table_hbm.at[idx_v.at[pl.ds(0, CL)]],
              rows_v.at[slot],
              sems.at[slot],
          ).wait()
          return c

        lax.fori_loop(0, CH, db, 0)

        def body(r, acc):
          out = []
          for j in range(NV):
            out.append(acc[j] + rows_v[slot, r, pl.ds(j * 16, 16)])
          for j in range(NV):
            rows_v[slot, r, pl.ds(j * 16, 16)] = zero
          return tuple(out)

        acc = lax.fori_loop(0, CL, body, (zero,) * NV)
        for j in range(NV):
          acc_v[si, pl.ds(j * 16, 16)] = acc[j]

      @pl.when(i < SPW)
      def _():
        def fb(ch, c):
          pltpu.async_copy(
              table_hbm.at[idx_v.at[pl.ds((i * CH + ch) * CL, CL)]],
              rows_v.at[slot],
              sems.at[slot],
              add=True,
          )
          return c

        lax.fori_loop(0, CH, fb, 0)

      return carry

    lax.fori_loop(0, SPW + NSLOT, step, 0)
    pltpu.sync_copy(acc_v, out_hbm.at[pl.ds(base, SPW)])

  return k(x3, table)


def _mlp_body(rep_ref, len_ref, w1t_ref, b1_ref, w2t_ref, b2_ref, out_ref):
  rep = rep_ref[...] / len_ref[...]
  h = jnp.dot(rep, w1t_ref[...], preferred_element_type=jnp.float32)
  h = jnp.maximum(h + b1_ref[...], 0.0)
  out = jnp.dot(h, w2t_ref[...], preferred_element_type=jnp.float32)
  out_ref[...] = out + b2_ref[...]


def kernel(x, lengths, table, W1, b1, W2, b2):
  B, L = x.shape
  D = table.shape[1]
  H = W1.shape[0]
  O = W2.shape[0]

  CL = max((r for r in range(8, 65, 8) if L % r == 0), default=L)
  CH = L // CL
  x3 = x.reshape(B * L)

  rep_sum = _gather_sum(x3, table, B, CH, CL, D)

  lens = lengths.astype(jnp.float32).reshape(B, 1)
  logits = pl.pallas_call(
      _mlp_body,
      out_shape=jax.ShapeDtypeStruct((B, O), jnp.float32),
  )(rep_sum, lens, W1.T, b1.reshape(1, H), W2.T, b2.reshape(1, O))
  return logits
